# Initial kernel scaffold; baseline (speedup 1.0000x reference)
import jax
import jax.numpy as jnp
from jax.experimental import pallas as pl


def kernel(x, y, W_lin, b_lin, Wg, W1, b1, W2, b2):
    raise NotImplementedError("write your pallas kernel here")


# speedup vs baseline: 1.2903x; 1.2903x over previous
import functools

import jax
import jax.numpy as jnp
from jax.experimental import pallas as pl
from jax.experimental.pallas import tpu as pltpu

B, S, D = 1, 2048, 768
E, K, FF = 8, 2, 3072
T = B * S
NFF = 2
FFB = FF // NFF


def _moe_kernel(x_ref, wlin_ref, blin_ref, wg_ref, w1_ref, b1_ref,
                w2_ref, b2_ref, y_ref, out_ref,
                lin_scr, comb_scr, acc_scr, xsum_scr):
    e = pl.program_id(0)
    f = pl.program_id(1)

    @pl.when(jnp.logical_and(e == 0, f == 0))
    def _prologue():
        xb = x_ref[...]
        lin = jax.lax.dot(xb.astype(jnp.bfloat16),
                          wlin_ref[...].astype(jnp.bfloat16),
                          preferred_element_type=jnp.float32)
        lin = lin + blin_ref[...]
        lin_scr[...] = lin.astype(jnp.bfloat16)
        xsum_scr[...] = jnp.sum(xb, axis=0, keepdims=True)
        logits = jax.lax.dot(lin, wg_ref[...],
                             preferred_element_type=jnp.float32)
        m = jnp.max(logits, axis=1, keepdims=True)
        p = jnp.exp(logits - m)
        p = p / jnp.sum(p, axis=1, keepdims=True)
        eids = jax.lax.broadcasted_iota(jnp.int32, (T, E), 1)
        v1 = jnp.max(p, axis=1, keepdims=True)
        i1 = jnp.min(jnp.where(p == v1, eids, E), axis=1, keepdims=True)
        p2 = jnp.where(eids == i1, -1.0, p)
        v2 = jnp.max(p2, axis=1, keepdims=True)
        i2 = jnp.min(jnp.where(p2 == v2, eids, E), axis=1, keepdims=True)
        den = v1 + v2
        comb_scr[...] = (jnp.where(eids == i1, v1 / den, 0.0)
                         + jnp.where(eids == i2, v2 / den, 0.0))
        acc_scr[...] = jnp.zeros_like(acc_scr)

    lin_bf = lin_scr[...]
    w1 = w1_ref[0].astype(jnp.bfloat16)
    h = jax.lax.dot(lin_bf, w1, preferred_element_type=jnp.float32)
    h = jax.nn.gelu(h + b1_ref[0])
    w2 = w2_ref[0].astype(jnp.bfloat16)
    eo = jax.lax.dot(h.astype(jnp.bfloat16), w2,
                     preferred_element_type=jnp.float32)
    eids = jax.lax.broadcasted_iota(jnp.int32, (1, E), 1)
    sel = jnp.sum(comb_scr[...] * (eids == e), axis=1, keepdims=True)
    contrib = jnp.sum(sel * eo, axis=0, keepdims=True)

    @pl.when(f == 0)
    def _b2():
        acc_scr[...] += jnp.sum(sel) * b2_ref[0]

    acc_scr[...] += contrib

    @pl.when(jnp.logical_and(e == E - 1, f == NFF - 1))
    def _epilogue():
        sent = (xsum_scr[...] + acc_scr[...]) * (1.0 / T)
        mx = jnp.max(sent)
        lse = mx + jnp.log(jnp.sum(jnp.exp(sent - mx)))
        cls = jax.lax.broadcasted_iota(jnp.int32, (1, D), 1)
        picked = jnp.sum(jnp.where(cls == y_ref[0, 0], sent, 0.0))
        out_ref[...] = jnp.broadcast_to(lse - picked, (1, 1))


@jax.jit
def _run(x, y, W_lin, b_lin, Wg, W1, b1, W2, b2):
    x2 = x.reshape(T, D)
    y32 = y.astype(jnp.int32).reshape(1, 1)
    grid = (E, NFF)
    out = pl.pallas_call(
        _moe_kernel,
        grid=grid,
        in_specs=[
            pl.BlockSpec((T, D), lambda e, f: (0, 0)),
            pl.BlockSpec((D, D), lambda e, f: (0, 0)),
            pl.BlockSpec((1, D), lambda e, f: (0, 0)),
            pl.BlockSpec((D, E), lambda e, f: (0, 0)),
            pl.BlockSpec((1, D, FFB), lambda e, f: (e, 0, f)),
            pl.BlockSpec((1, 1, FFB), lambda e, f: (e, 0, f)),
            pl.BlockSpec((1, FFB, D), lambda e, f: (e, f, 0)),
            pl.BlockSpec((1, 1, D), lambda e, f: (e, 0, 0)),
            pl.BlockSpec(memory_space=pltpu.SMEM),
        ],
        out_specs=pl.BlockSpec((1, 1), lambda e, f: (0, 0)),
        out_shape=jax.ShapeDtypeStruct((1, 1), jnp.float32),
        scratch_shapes=[
            pltpu.VMEM((T, D), jnp.bfloat16),
            pltpu.VMEM((T, E), jnp.float32),
            pltpu.VMEM((1, D), jnp.float32),
            pltpu.VMEM((1, D), jnp.float32),
        ],
        compiler_params=pltpu.CompilerParams(
            dimension_semantics=("arbitrary", "arbitrary"),
        ),
    )(x2, W_lin, b_lin.reshape(1, D), Wg, W1, b1.reshape(E, 1, FF),
      W2, b2.reshape(E, 1, D), y32)
    return out[0, 0]


def kernel(x, y, W_lin, b_lin, Wg, W1, b1, W2, b2):
    return _run(x, y, W_lin, b_lin, Wg, W1, b1, W2, b2)
